# Initial kernel scaffold; baseline (speedup 1.0000x reference)
import jax
import jax.numpy as jnp
from jax.experimental import pallas as pl


def kernel(hidden_states, Wq, Wk, Wv, W1, b1, W2, b2):
    raise NotImplementedError("write your pallas kernel here")


# speedup vs baseline: 1.3337x; 1.3337x over previous
import functools
import math

import jax
import jax.numpy as jnp
from jax import lax
from jax.experimental import pallas as pl
from jax.experimental.pallas import tpu as pltpu
from jax.experimental.pallas import tpu_sc as plsc

B, T, C = 2, 2048, 768
E, TOPK, SPAN, H = 8, 2, 64, 12
DH = C // H
S = T // SPAN
NSP = B * S
NTILE = NSP // 16


def _gating_body(x_ref, w1_ref, b1_ref, w2_ref, b2_ref, lg_ref):
    pooled = jnp.mean(x_ref[...], axis=1)
    g = lax.dot_general(pooled, w1_ref[...], (((1,), (1,)), ((), ())),
                        preferred_element_type=jnp.float32)
    g = jnp.maximum(g + b1_ref[...], 0.0)
    logits = lax.dot_general(g, w2_ref[...], (((1,), (1,)), ((), ())),
                             preferred_element_type=jnp.float32)
    logits = logits + b2_ref[...]
    lgT = logits.T
    for w in range(NTILE):
        lg_ref[w] = lgT[:, 16 * w:16 * w + 16]


def _sc_gate_body(lg_hbm, ti_hbm, sm_hbm, lg_v, ti_v, sm_v):
    wid = lax.axis_index("s") * 2 + lax.axis_index("c")

    @pl.when(wid < NTILE)
    def _():
        pltpu.sync_copy(lg_hbm.at[wid], lg_v)
        neg = jnp.full((16,), -jnp.inf, jnp.float32)
        m1 = neg
        m2 = neg
        i1 = jnp.zeros((16,), jnp.int32)
        i2 = jnp.zeros((16,), jnp.int32)
        for e in range(E):
            x = lg_v[e, :]
            ev = jnp.full((16,), e, jnp.int32)
            gt1 = x > m1
            gt2 = x > m2
            m2 = jnp.where(gt1, m1, jnp.where(gt2, x, m2))
            i2 = jnp.where(gt1, i1, jnp.where(gt2, ev, i2))
            m1 = jnp.where(gt1, x, m1)
            i1 = jnp.where(gt1, ev, i1)
        d = jnp.exp(m2 - m1)
        denom = 1.0 + d
        ti_v[0, :] = i1
        ti_v[1, :] = i2
        sm_v[0, :] = 1.0 / denom
        sm_v[1, :] = d / denom
        pltpu.sync_copy(ti_v, ti_hbm.at[wid])
        pltpu.sync_copy(sm_v, sm_hbm.at[wid])


P = 4


def _main_body(ti_ref, sm_ref, x_ref, wqkv_ref, out_ref):
    g = pl.program_id(0)
    dn = (((1,), (1,)), ((), ()))
    scale = 1.0 / math.sqrt(DH)
    for pi in range(P):
        span = g * P + pi
        w = span // 16
        j = span % 16
        t0 = ti_ref[w * 32 + j]
        t1 = ti_ref[w * 32 + 16 + j]
        s0 = sm_ref[w * 32 + j]
        s1 = sm_ref[w * 32 + 16 + j]
        x = x_ref[pi]

        a = lax.dot_general(x, wqkv_ref[t0], dn,
                            preferred_element_type=jnp.float32)
        bb = lax.dot_general(x, wqkv_ref[t1], dn,
                             preferred_element_type=jnp.float32)
        qkv = s0 * a + s1 * bb
        q = qkv[:, 0:C]
        k = qkv[:, C:2 * C]
        v = qkv[:, 2 * C:3 * C]
        qb = (q * scale).astype(jnp.bfloat16)
        kb = k.astype(jnp.bfloat16)
        vb = v.astype(jnp.bfloat16)
        G = 4
        for g0 in range(0, H, G):
            scores = jnp.concatenate(
                [lax.dot_general(qb[:, h * DH:(h + 1) * DH],
                                 kb[:, h * DH:(h + 1) * DH],
                                 (((1,), (1,)), ((), ())),
                                 preferred_element_type=jnp.float32)
                 for h in range(g0, g0 + G)], axis=1)
            s3 = scores.reshape(SPAN, G, SPAN)
            m = jnp.max(s3, axis=2, keepdims=True)
            p = jnp.exp(s3 - m)
            p = p / jnp.sum(p, axis=2, keepdims=True)
            pb = p.reshape(SPAN, G * SPAN).astype(jnp.bfloat16)
            for hi in range(G):
                h = g0 + hi
                lo = h * DH
                out_ref[pi, :, lo:lo + DH] = lax.dot_general(
                    pb[:, hi * SPAN:(hi + 1) * SPAN], vb[:, lo:lo + DH],
                    (((1,), (0,)), ((), ())),
                    preferred_element_type=jnp.float32)


def _sc_gate(logits_blk):
    return pl.kernel(
        _sc_gate_body,
        out_type=(
            jax.ShapeDtypeStruct((NTILE, TOPK, 16), jnp.int32),
            jax.ShapeDtypeStruct((NTILE, TOPK, 16), jnp.float32),
        ),
        mesh=plsc.VectorSubcoreMesh(core_axis_name="c", subcore_axis_name="s"),
        scratch_types=[
            pltpu.VMEM((E, 16), jnp.float32),
            pltpu.VMEM((TOPK, 16), jnp.int32),
            pltpu.VMEM((TOPK, 16), jnp.float32),
        ],
    )(logits_blk)


@jax.jit
def kernel(hidden_states, Wq, Wk, Wv, W1, b1, W2, b2):
    x = hidden_states.reshape(NSP, SPAN, C)

    logits_blk = pl.pallas_call(
        _gating_body,
        out_shape=jax.ShapeDtypeStruct((NTILE, E, 16), jnp.float32),
    )(x, W1, b1.reshape(1, 128), W2, b2.reshape(1, E))

    ti_blk, sm_blk = _sc_gate(logits_blk)

    xb = x.astype(jnp.bfloat16)
    wqkv = jnp.concatenate([Wq, Wk, Wv], axis=1).astype(jnp.bfloat16)
    out = pl.pallas_call(
        _main_body,
        grid_spec=pltpu.PrefetchScalarGridSpec(
            num_scalar_prefetch=2,
            grid=(NSP // P,),
            in_specs=[
                pl.BlockSpec((P, SPAN, C), lambda s, tir, smr: (s, 0, 0)),
                pl.BlockSpec((E, 3 * C, C), lambda s, tir, smr: (0, 0, 0)),
            ],
            out_specs=pl.BlockSpec((P, SPAN, C), lambda s, tir, smr: (s, 0, 0)),
        ),
        out_shape=jax.ShapeDtypeStruct((NSP, SPAN, C), jnp.float32),
        compiler_params=pltpu.CompilerParams(
            dimension_semantics=("arbitrary",),
            vmem_limit_bytes=100 * 1024 * 1024,
        ),
    )(ti_blk.reshape(NTILE * TOPK * 16), sm_blk.reshape(NTILE * TOPK * 16),
      xb, wqkv)

    return out.reshape(B, T, C)
